# baseline (speedup 1.0000x reference)
import functools

import jax
import jax.numpy as jnp
from jax import lax
from jax.experimental import pallas as pl
from jax.experimental.pallas import tpu as pltpu
from jax.experimental.pallas import tpu_sc as plsc

_NC, _NS = 2, 16
_NW = _NC * _NS
_CH = 128
_NB = 4
_RW = 128
_CP = 32


def _widen(tbl):
    return jnp.pad(tbl, ((0, 0), (0, tbl.shape[1])))


def _ring_pipeline(buf, z_hbm, gsem, ssem):
    def wait_gather(slot):
        pltpu.make_async_copy(z_hbm, buf.at[slot], gsem).wait()

    def wait_consume(slot):
        pltpu.make_async_copy(z_hbm, buf.at[slot], ssem).wait()

    def pipeline(tbl, idx_v, nch, consume):
        for j in range(min(_NB, nch)):
            pltpu.async_copy(tbl.at[idx_v.at[j]], buf.at[j], gsem)

        def body(k, carry):
            slot = lax.rem(k, _NB)
            wait_gather(slot)
            consume(k, slot)
            nk = k + _NB

            @pl.when(nk < nch)
            def _():
                wait_consume(slot)
                pltpu.async_copy(tbl.at[idx_v.at[nk]], buf.at[slot], gsem)

            return carry

        lax.fori_loop(0, nch, body, 0)
        for _ in range(min(_NB, nch)):
            wait_consume(0)

    return pipeline


def _sc_pool(word_ids, gram_ids, wt128, gt128):
    B, Lw = word_ids.shape
    bpw = B // _NW
    nwch = (B * Lw) // (_NW * _CH)
    rows_per_sc = _NS * bpw

    wid3 = word_ids.reshape(_NW, nwch, _CH).astype(jnp.int32)
    gid3 = gram_ids.reshape(_NW, nwch, _CH).astype(jnp.int32)
    smap = ((jnp.arange(B * Lw, dtype=jnp.int32) // Lw) % rows_per_sc).reshape(
        _NW, nwch, _CH)
    zrows = jnp.zeros((_CH, _RW), jnp.float32)

    mesh = plsc.VectorSubcoreMesh(core_axis_name="c", subcore_axis_name="s")

    @functools.partial(
        pl.kernel,
        out_type=jax.ShapeDtypeStruct((B, _RW), jnp.float32),
        mesh=mesh,
        scratch_types=[
            pltpu.VMEM((nwch, _CH), jnp.int32),
            pltpu.VMEM((nwch, _CH), jnp.int32),
            pltpu.VMEM((nwch, _CH), jnp.int32),
            pltpu.VMEM((_NB, _CH, _RW), jnp.float32),
            pltpu.VMEM_SHARED((rows_per_sc, _RW), jnp.float32),
            pltpu.SemaphoreType.DMA,
            pltpu.SemaphoreType.DMA,
        ],
    )
    def sc_kern(wt_hbm, gt_hbm, wid_hbm, gid_hbm, smap_hbm, z_hbm, pooled_hbm,
                widx_v, gidx_v, map_v, buf, pooled_sh, gsem, ssem):
        c = lax.axis_index("c")
        s = lax.axis_index("s")
        w = c * _NS + s
        pipeline = _ring_pipeline(buf, z_hbm, gsem, ssem)

        pltpu.sync_copy(z_hbm, pooled_sh.at[pl.ds(s * bpw, bpw)])
        pltpu.sync_copy(smap_hbm.at[w], map_v)
        pltpu.sync_copy(wid_hbm.at[w], widx_v)
        pltpu.sync_copy(gid_hbm.at[w], gidx_v)

        def pool_consume(k, slot):
            pltpu.async_copy(buf.at[slot], pooled_sh.at[map_v.at[k]], ssem,
                             add=True)

        pipeline(wt_hbm, widx_v, nwch, pool_consume)
        pipeline(gt_hbm, gidx_v, nwch, pool_consume)
        pltpu.sync_copy(pooled_sh.at[pl.ds(s * bpw, bpw)],
                        pooled_hbm.at[pl.ds(w * bpw, bpw)])

    return sc_kern(wt128, gt128, wid3, gid3, smap, zrows)


def _sc_ent(ent_ids, et128):
    B, C = ent_ids.shape
    bpw = B // _NW
    nech = (B * _CP) // (_NW * _CH)
    eidp = jnp.zeros((B, _CP), jnp.int32).at[:, :C].set(
        ent_ids.astype(jnp.int32)).reshape(_NW, nech, _CH)
    zrows = jnp.zeros((_CH, _RW), jnp.float32)

    mesh = plsc.VectorSubcoreMesh(core_axis_name="c", subcore_axis_name="s")

    @functools.partial(
        pl.kernel,
        out_type=jax.ShapeDtypeStruct((B * _CP, _RW), jnp.float32),
        mesh=mesh,
        scratch_types=[
            pltpu.VMEM((nech, _CH), jnp.int32),
            pltpu.VMEM((_NB, _CH, _RW), jnp.float32),
            pltpu.SemaphoreType.DMA,
            pltpu.SemaphoreType.DMA,
        ],
    )
    def sc_kern(et_hbm, eid_hbm, z_hbm, ee_hbm, eidx_v, buf, gsem, ssem):
        c = lax.axis_index("c")
        s = lax.axis_index("s")
        w = c * _NS + s
        pipeline = _ring_pipeline(buf, z_hbm, gsem, ssem)
        pltpu.sync_copy(eid_hbm.at[w], eidx_v)

        def ent_consume(k, slot):
            pltpu.async_copy(buf.at[slot],
                             ee_hbm.at[pl.ds((w * nech + k) * _CH, _CH)], ssem)

        pipeline(et_hbm, eidx_v, nech, ent_consume)

    return sc_kern(et128, eidp, zrows)


def _tc_score(pooled, ee3, W, b, C, inv_scale):
    B = pooled.shape[0]
    D = W.shape[0]
    BB = 512

    def body(p_ref, w_ref, b_ref, e_ref, o_ref):
        ctx = lax.dot_general(p_ref[...][:, :D], w_ref[...],
                              (((1,), (1,)), ((), ())),
                              preferred_element_type=jnp.float32)
        ctx = ctx * inv_scale + b_ref[...]
        o_ref[...] = jnp.sum(e_ref[...][:, :C, :D] * ctx[:, None, :], axis=-1)

    return pl.pallas_call(
        body,
        grid=(B // BB,),
        in_specs=[
            pl.BlockSpec((BB, _RW), lambda i: (i, 0)),
            pl.BlockSpec((D, D), lambda i: (0, 0)),
            pl.BlockSpec((1, D), lambda i: (0, 0)),
            pl.BlockSpec((BB, _CP, _RW), lambda i: (i, 0, 0)),
        ],
        out_specs=pl.BlockSpec((BB, C), lambda i: (i, 0)),
        out_shape=jax.ShapeDtypeStruct((B, C), jnp.float32),
    )(pooled, W, b.reshape(1, D), ee3)


def kernel(word_ids, gram_ids, ent_ids, word_table, gram_table, ent_table, W, b):
    B, C = ent_ids.shape
    wt128 = _widen(word_table)
    gt128 = _widen(gram_table)
    et128 = _widen(ent_table)
    pooled = _sc_pool(word_ids, gram_ids, wt128, gt128)
    ee2 = _sc_ent(ent_ids, et128)
    ee3 = ee2.reshape(B, _CP, _RW)
    return _tc_score(pooled, ee3, W, b, C, 1.0 / word_ids.shape[1])


# speedup vs baseline: 2.4671x; 2.4671x over previous
import functools

import jax
import jax.numpy as jnp
from jax import lax
from jax.experimental import pallas as pl
from jax.experimental.pallas import tpu as pltpu
from jax.experimental.pallas import tpu_sc as plsc

_NC, _NS = 2, 16
_NW = _NC * _NS
_CH = 128
_NB = 8


def _sc_gather_pool(word_ids, gram_ids, ent_ids, word_table, gram_table, ent_table):
    B, Lw = word_ids.shape
    _, Lg = gram_ids.shape
    _, C = ent_ids.shape
    D = word_table.shape[1]
    assert Lw == Lg, "shared scatter map assumes equal pooling widths"
    bpw = B // _NW
    nwch = (B * Lw) // (_NW * _CH)
    nech = (B * C) // (_NW * _CH)
    rows_per_sc = _NS * bpw

    wid3 = word_ids.reshape(_NW, nwch, _CH).astype(jnp.int32)
    gid3 = gram_ids.reshape(_NW, nwch, _CH).astype(jnp.int32)
    eid3 = ent_ids.reshape(_NW, nech, _CH).astype(jnp.int32)
    smap = ((jnp.arange(B * Lw, dtype=jnp.int32) // Lw) % rows_per_sc).reshape(
        _NW, nwch, _CH)
    zrows = jnp.zeros((bpw, D), jnp.float32)

    mesh = plsc.VectorSubcoreMesh(core_axis_name="c", subcore_axis_name="s")

    @functools.partial(
        pl.kernel,
        out_type=(jax.ShapeDtypeStruct((B, D), jnp.float32),
                  jax.ShapeDtypeStruct((B * C, D), jnp.float32)),
        mesh=mesh,
        scratch_types=[
            pltpu.VMEM((nwch, _CH), jnp.int32),
            pltpu.VMEM((nwch, _CH), jnp.int32),
            pltpu.VMEM((nech, _CH), jnp.int32),
            pltpu.VMEM((nwch, _CH), jnp.int32),
            pltpu.VMEM((_NB, _CH, D), jnp.float32),
            pltpu.VMEM_SHARED((rows_per_sc, D), jnp.float32),
            pltpu.SemaphoreType.DMA,
            pltpu.SemaphoreType.DMA,
        ],
        compiler_params=pltpu.CompilerParams(use_tc_tiling_on_sc=False),
    )
    def sc_kern(wt_hbm, gt_hbm, et_hbm, wid_hbm, gid_hbm, eid_hbm, smap_hbm,
                z_hbm, pooled_hbm, ee_hbm, widx_v, gidx_v, eidx_v, map_v, buf,
                pooled_sh, gsem, ssem):
        c = lax.axis_index("c")
        s = lax.axis_index("s")
        w = c * _NS + s

        def wait_gather(slot):
            pltpu.make_async_copy(z_hbm, buf.at[slot], gsem).wait()

        def wait_consume(slot):
            pltpu.make_async_copy(z_hbm, buf.at[slot], ssem).wait()

        def pipeline(tbl, idx_v, nch, consume):
            for j in range(min(_NB, nch)):
                pltpu.async_copy(tbl.at[idx_v.at[j]], buf.at[j], gsem)

            def body(k, carry):
                slot = lax.rem(k, _NB)
                wait_gather(slot)
                consume(k, slot)
                nk = k + _NB

                @pl.when(nk < nch)
                def _():
                    wait_consume(slot)
                    pltpu.async_copy(tbl.at[idx_v.at[nk]], buf.at[slot], gsem)

                return carry

            lax.fori_loop(0, nch, body, 0)
            for _ in range(min(_NB, nch)):
                wait_consume(0)

        pltpu.sync_copy(z_hbm, pooled_sh.at[pl.ds(s * bpw, bpw)])
        pltpu.sync_copy(smap_hbm.at[w], map_v)
        pltpu.sync_copy(wid_hbm.at[w], widx_v)
        pltpu.sync_copy(gid_hbm.at[w], gidx_v)
        pltpu.sync_copy(eid_hbm.at[w], eidx_v)

        def pool_consume(k, slot):
            pltpu.async_copy(buf.at[slot], pooled_sh.at[map_v.at[k]], ssem,
                             add=True)

        def ent_consume(k, slot):
            pltpu.async_copy(buf.at[slot],
                             ee_hbm.at[pl.ds((w * nech + k) * _CH, _CH)], ssem)

        pipeline(wt_hbm, widx_v, nwch, pool_consume)
        pipeline(gt_hbm, gidx_v, nwch, pool_consume)
        pltpu.sync_copy(pooled_sh.at[pl.ds(s * bpw, bpw)],
                        pooled_hbm.at[pl.ds(w * bpw, bpw)])
        pipeline(et_hbm, eidx_v, nech, ent_consume)

    return sc_kern(word_table, gram_table, ent_table, wid3, gid3, eid3, smap,
                   zrows)


def _tc_score(pooled, ee, W, b, inv_scale):
    B, D = pooled.shape
    C = ee.shape[0] // B
    ee3 = ee.reshape(B, C, D)
    BB = 512

    def body(p_ref, w_ref, b_ref, e_ref, o_ref):
        ctx = lax.dot_general(p_ref[...], w_ref[...], (((1,), (1,)), ((), ())),
                              preferred_element_type=jnp.float32)
        ctx = ctx * inv_scale + b_ref[...]
        o_ref[...] = jnp.sum(e_ref[...] * ctx[:, None, :], axis=-1)

    return pl.pallas_call(
        body,
        grid=(B // BB,),
        in_specs=[
            pl.BlockSpec((BB, D), lambda i: (i, 0)),
            pl.BlockSpec((D, D), lambda i: (0, 0)),
            pl.BlockSpec((1, D), lambda i: (0, 0)),
            pl.BlockSpec((BB, C, D), lambda i: (i, 0, 0)),
        ],
        out_specs=pl.BlockSpec((BB, C), lambda i: (i, 0)),
        out_shape=jax.ShapeDtypeStruct((B, C), jnp.float32),
    )(pooled, W, b.reshape(1, D), ee3)


def kernel(word_ids, gram_ids, ent_ids, word_table, gram_table, ent_table, W, b):
    pooled, ee = _sc_gather_pool(word_ids, gram_ids, ent_ids,
                                 word_table, gram_table, ent_table)
    return _tc_score(pooled, ee, W, b, 1.0 / word_ids.shape[1])


# speedup vs baseline: 2.4841x; 1.0069x over previous
import functools

import jax
import jax.numpy as jnp
from jax import lax
from jax.experimental import pallas as pl
from jax.experimental.pallas import tpu as pltpu
from jax.experimental.pallas import tpu_sc as plsc

_NC, _NS = 2, 16
_NW = _NC * _NS
_CH = 128
_NB = 8


def _sc_gather_pool(word_ids, gram_ids, ent_ids, word_table, gram_table, ent_table):
    B, Lw = word_ids.shape
    _, Lg = gram_ids.shape
    _, C = ent_ids.shape
    D = word_table.shape[1]
    assert Lw == Lg, "shared scatter map assumes equal pooling widths"
    bpw = B // _NW
    nwch = (B * Lw) // (_NW * _CH)
    nech = (B * C) // (_NW * _CH)
    rows_per_sc = _NS * bpw

    wid3 = word_ids.reshape(_NW, nwch, _CH).astype(jnp.int32)
    gid3 = gram_ids.reshape(_NW, nwch, _CH).astype(jnp.int32)
    eid3 = ent_ids.reshape(_NW, nech, _CH).astype(jnp.int32)
    smap = ((jnp.arange(B * Lw, dtype=jnp.int32) // Lw) % rows_per_sc).reshape(
        _NW, nwch, _CH)
    zrows = jnp.zeros((bpw, D), jnp.float32)

    mesh = plsc.VectorSubcoreMesh(core_axis_name="c", subcore_axis_name="s")

    @functools.partial(
        pl.kernel,
        out_type=(jax.ShapeDtypeStruct((B, D), jnp.float32),
                  jax.ShapeDtypeStruct((B * C, D), jnp.float32)),
        mesh=mesh,
        scratch_types=[
            pltpu.VMEM((nwch, _CH), jnp.int32),
            pltpu.VMEM((nwch, _CH), jnp.int32),
            pltpu.VMEM((nech, _CH), jnp.int32),
            pltpu.VMEM((nwch, _CH), jnp.int32),
            pltpu.VMEM((_NB, _CH, D), jnp.float32),
            pltpu.VMEM_SHARED((rows_per_sc, D), jnp.float32),
            pltpu.SemaphoreType.DMA,
            pltpu.SemaphoreType.DMA,
        ],
        compiler_params=pltpu.CompilerParams(use_tc_tiling_on_sc=False),
    )
    def sc_kern(wt_hbm, gt_hbm, et_hbm, wid_hbm, gid_hbm, eid_hbm, smap_hbm,
                z_hbm, pooled_hbm, ee_hbm, widx_v, gidx_v, eidx_v, map_v, buf,
                pooled_sh, gsem, ssem):
        c = lax.axis_index("c")
        s = lax.axis_index("s")
        w = c * _NS + s

        def wait_gather(slot):
            pltpu.make_async_copy(z_hbm, buf.at[slot], gsem).wait()

        def wait_consume(slot):
            pltpu.make_async_copy(z_hbm, buf.at[slot], ssem).wait()

        def pipeline(tbl, idx_v, nch, consume):
            for j in range(min(_NB, nch)):
                pltpu.async_copy(tbl.at[idx_v.at[j]], buf.at[j], gsem)

            def body(k, carry):
                slot = lax.rem(k, _NB)
                wait_gather(slot)
                consume(k, slot)
                nk = k + _NB

                @pl.when(nk < nch)
                def _():
                    wait_consume(slot)
                    pltpu.async_copy(tbl.at[idx_v.at[nk]], buf.at[slot], gsem)

                return carry

            lax.fori_loop(0, nch, body, 0)
            for _ in range(min(_NB, nch)):
                wait_consume(0)

        pltpu.sync_copy(z_hbm, pooled_sh.at[pl.ds(s * bpw, bpw)])
        pltpu.sync_copy(smap_hbm.at[w], map_v)
        pltpu.sync_copy(wid_hbm.at[w], widx_v)
        pltpu.sync_copy(gid_hbm.at[w], gidx_v)
        pltpu.sync_copy(eid_hbm.at[w], eidx_v)

        def pool_consume(k, slot):
            pltpu.async_copy(buf.at[slot], pooled_sh.at[map_v.at[k]], ssem,
                             add=True)

        def ent_consume(k, slot):
            pltpu.async_copy(buf.at[slot],
                             ee_hbm.at[pl.ds((w * nech + k) * _CH, _CH)], ssem)

        pipeline(wt_hbm, widx_v, nwch, pool_consume)
        pipeline(gt_hbm, gidx_v, nwch, pool_consume)
        pltpu.sync_copy(pooled_sh.at[pl.ds(s * bpw, bpw)],
                        pooled_hbm.at[pl.ds(w * bpw, bpw)])
        pipeline(et_hbm, eidx_v, nech, ent_consume)

    return sc_kern(word_table, gram_table, ent_table, wid3, gid3, eid3, smap,
                   zrows)


def _tc_score(pooled, ee, W, b, inv_scale):
    B, D = pooled.shape
    C = ee.shape[0] // B
    BB = 512

    def body(p_ref, w_ref, b_ref, e_ref, o_ref):
        ctx = lax.dot_general(p_ref[...], w_ref[...], (((1,), (1,)), ((), ())),
                              preferred_element_type=jnp.float32)
        ctx = ctx * inv_scale + b_ref[...]
        e3 = e_ref[...].reshape(BB, C, D)
        o_ref[...] = jnp.sum(e3 * ctx[:, None, :], axis=-1)

    return pl.pallas_call(
        body,
        grid=(B // BB,),
        in_specs=[
            pl.BlockSpec((BB, D), lambda i: (i, 0)),
            pl.BlockSpec((D, D), lambda i: (0, 0)),
            pl.BlockSpec((1, D), lambda i: (0, 0)),
            pl.BlockSpec((BB * C, D), lambda i: (i, 0)),
        ],
        out_specs=pl.BlockSpec((BB, C), lambda i: (i, 0)),
        out_shape=jax.ShapeDtypeStruct((B, C), jnp.float32),
    )(pooled, W, b.reshape(1, D), ee)


def kernel(word_ids, gram_ids, ent_ids, word_table, gram_table, ent_table, W, b):
    pooled, ee = _sc_gather_pool(word_ids, gram_ids, ent_ids,
                                 word_table, gram_table, ent_table)
    return _tc_score(pooled, ee, W, b, 1.0 / word_ids.shape[1])
